# baseline (speedup 1.0000x reference)
import functools

import jax
import jax.numpy as jnp
from jax import lax
from jax.experimental import pallas as pl
from jax.experimental.pallas import tpu as pltpu

NUM_LAYERS = 4
NUM_EMBEDDINGS = 1024
EMBEDDING_DIM = 64
COMMITMENT_COST = 0.25

TILE = 2048


def _rvq_tile(x_ref, emb_ref, q_ref, idx_ref, loss_ref, en_ref, e2_ref):
    i = pl.program_id(0)

    @pl.when(i == 0)
    def _precompute():
        loss_ref[...] = jnp.zeros((1, 1, 1), jnp.float32)
        for l in range(NUM_LAYERS):
            emb = emb_ref[l]
            en_ref[l, :] = jnp.sum(emb * emb, axis=1)
            e2_ref[l] = emb + emb

    r = x_ref[...]
    qacc = jnp.zeros_like(r)
    loss_acc = jnp.float32(0.0)
    for l in range(NUM_LAYERS):
        emb = emb_ref[l]
        e_norms = en_ref[l, :]
        r_norms = jnp.sum(r * r, axis=1, keepdims=True)
        dots2 = lax.dot_general(
            r, e2_ref[l], (((1,), (1,)), ((), ())),
            preferred_element_type=jnp.float32,
        )
        dist = (r_norms + e_norms[None, :]) - dots2
        dmin8 = dist[:, 0:128]
        for k in range(1, 8):
            dmin8 = jnp.minimum(dmin8, dist[:, 128 * k : 128 * (k + 1)])
        mins = jnp.min(dmin8, axis=1, keepdims=True)
        jidx_f = lax.broadcasted_iota(
            jnp.int32, (1, NUM_EMBEDDINGS), 1
        ).astype(jnp.float32)
        wsel = jnp.where(dist == mins, jidx_f, jnp.float32(NUM_EMBEDDINGS))
        wmin8 = wsel[:, 0:128]
        for k in range(1, 8):
            wmin8 = jnp.minimum(wmin8, wsel[:, 128 * k : 128 * (k + 1)])
        idx_f = jnp.min(wmin8, axis=1)
        idx = idx_f.astype(jnp.int32)
        onehot = (jidx_f == idx_f[:, None]).astype(jnp.float32)
        q = lax.dot_general(
            onehot, emb, (((1,), (0,)), ((), ())),
            preferred_element_type=jnp.float32,
        )
        loss_acc += jnp.sum((q - r) * (q - r))
        q_ste = r + (q - r)
        r = r - q_ste
        qacc = qacc + q_ste
        idx_ref[:, l : l + 1] = idx[:, None]
    q_ref[...] = qacc
    loss_ref[...] += loss_acc.reshape(1, 1, 1)


@functools.partial(jax.jit, static_argnames=())
def kernel(x, embeddings):
    B, S, D = x.shape
    n_tokens = B * S
    x_flat = x.reshape(n_tokens, D)
    grid = (n_tokens // TILE,)

    q_flat, idx_lt, loss = pl.pallas_call(
        _rvq_tile,
        grid=grid,
        in_specs=[
            pl.BlockSpec((TILE, D), lambda i: (i, 0)),
            pl.BlockSpec((NUM_LAYERS, NUM_EMBEDDINGS, D), lambda i: (0, 0, 0)),
        ],
        out_specs=[
            pl.BlockSpec((TILE, D), lambda i: (i, 0)),
            pl.BlockSpec((TILE, NUM_LAYERS), lambda i: (i, 0)),
            pl.BlockSpec((1, 1, 1), lambda i: (0, 0, 0)),
        ],
        out_shape=[
            jax.ShapeDtypeStruct((n_tokens, D), jnp.float32),
            jax.ShapeDtypeStruct((n_tokens, NUM_LAYERS), jnp.int32),
            jax.ShapeDtypeStruct((1, 1, 1), jnp.float32),
        ],
        scratch_shapes=[
            pltpu.VMEM((NUM_LAYERS, NUM_EMBEDDINGS), jnp.float32),
            pltpu.VMEM((NUM_LAYERS, NUM_EMBEDDINGS, EMBEDDING_DIM), jnp.float32),
        ],
    )(x_flat, embeddings)

    quantized_out = q_flat.reshape(B, S, D)
    losses = loss[0, 0, 0] * (COMMITMENT_COST / n_tokens / D)
    all_indices = idx_lt.reshape(B, S, NUM_LAYERS)
    return quantized_out, losses, all_indices


# speedup vs baseline: 1.0207x; 1.0207x over previous
import functools

import jax
import jax.numpy as jnp
from jax import lax
from jax.experimental import pallas as pl
from jax.experimental.pallas import tpu as pltpu

NUM_LAYERS = 4
NUM_EMBEDDINGS = 1024
EMBEDDING_DIM = 64
COMMITMENT_COST = 0.25

TILE = 2048


def _rvq_tile(x_ref, emb_ref, q_ref, idx_ref, loss_ref, en_ref, e2_ref):
    i = pl.program_id(0)

    @pl.when(i == 0)
    def _precompute():
        loss_ref[...] = jnp.zeros((1, 1, 1), jnp.float32)
        for l in range(NUM_LAYERS):
            emb = emb_ref[l]
            en_ref[l, :] = jnp.sum(emb * emb, axis=1)
            e2_ref[l] = emb + emb

    r = x_ref[...]
    qacc = jnp.zeros_like(r)
    loss_acc = jnp.float32(0.0)
    for l in range(NUM_LAYERS):
        emb = emb_ref[l]
        e_norms = en_ref[l, :]
        r_norms = jnp.sum(r * r, axis=1, keepdims=True)
        dots2 = lax.dot_general(
            r, e2_ref[l], (((1,), (1,)), ((), ())),
            preferred_element_type=jnp.float32,
        )
        dist = (r_norms + e_norms[None, :]) - dots2
        dmin8 = dist[:, 0:128]
        for k in range(1, 8):
            dmin8 = jnp.minimum(dmin8, dist[:, 128 * k : 128 * (k + 1)])
        mins = jnp.min(dmin8, axis=1, keepdims=True)
        jidx_f = lax.broadcasted_iota(
            jnp.int32, (1, NUM_EMBEDDINGS), 1
        ).astype(jnp.float32)
        wsel = jnp.where(dist == mins, jidx_f, jnp.float32(NUM_EMBEDDINGS))
        wmin8 = wsel[:, 0:128]
        for k in range(1, 8):
            wmin8 = jnp.minimum(wmin8, wsel[:, 128 * k : 128 * (k + 1)])
        idx_f = jnp.min(wmin8, axis=1)
        idx = idx_f.astype(jnp.int32)
        onehot = (jidx_f == idx_f[:, None]).astype(jnp.float32)
        q = lax.dot_general(
            onehot, emb, (((1,), (0,)), ((), ())),
            preferred_element_type=jnp.float32,
        )
        loss_acc += jnp.sum((q - r) * (q - r))
        q_ste = r + (q - r)
        r = r - q_ste
        qacc = qacc + q_ste
        idx_ref[l, :] = idx
    q_ref[...] = qacc
    loss_ref[...] += loss_acc.reshape(1, 1, 1)


@functools.partial(jax.jit, static_argnames=())
def kernel(x, embeddings):
    B, S, D = x.shape
    n_tokens = B * S
    x_flat = x.reshape(n_tokens, D)
    grid = (n_tokens // TILE,)

    q_flat, idx_lt, loss = pl.pallas_call(
        _rvq_tile,
        grid=grid,
        in_specs=[
            pl.BlockSpec((TILE, D), lambda i: (i, 0)),
            pl.BlockSpec((NUM_LAYERS, NUM_EMBEDDINGS, D), lambda i: (0, 0, 0)),
        ],
        out_specs=[
            pl.BlockSpec((TILE, D), lambda i: (i, 0)),
            pl.BlockSpec((NUM_LAYERS, TILE), lambda i: (0, i)),
            pl.BlockSpec((1, 1, 1), lambda i: (0, 0, 0)),
        ],
        out_shape=[
            jax.ShapeDtypeStruct((n_tokens, D), jnp.float32),
            jax.ShapeDtypeStruct((NUM_LAYERS, n_tokens), jnp.int32),
            jax.ShapeDtypeStruct((1, 1, 1), jnp.float32),
        ],
        scratch_shapes=[
            pltpu.VMEM((NUM_LAYERS, NUM_EMBEDDINGS), jnp.float32),
            pltpu.VMEM((NUM_LAYERS, NUM_EMBEDDINGS, EMBEDDING_DIM), jnp.float32),
        ],
    )(x_flat, embeddings)

    quantized_out = q_flat.reshape(B, S, D)
    losses = loss[0, 0, 0] * (COMMITMENT_COST / n_tokens / D)
    all_indices = idx_lt.T.reshape(B, S, NUM_LAYERS)
    return quantized_out, losses, all_indices
